# Initial kernel scaffold; baseline (speedup 1.0000x reference)
import jax
import jax.numpy as jnp
from jax.experimental import pallas as pl


def kernel(features, edge_index, edge_weight, W0, b0, W1, b1, W2, b2):
    raise NotImplementedError("write your pallas kernel here")


# speedup vs baseline: 3.3059x; 3.3059x over previous
import dataclasses
import functools

import jax
import jax.numpy as jnp
from jax import lax
from jax.experimental import pallas as pl
from jax.experimental.pallas import tpu as pltpu
from jax.experimental.pallas import tpu_sc as plsc

N = 10000
D = 128
DH = D // 2
NC = 2
NS = 16
LANES = 16
EB = 80

_SC_PARAMS = pltpu.CompilerParams(use_tc_tiling_on_sc=False)
if "needs_layout_passes" in pltpu.CompilerParams.__dataclass_fields__:
    _SC_PARAMS = dataclasses.replace(_SC_PARAMS, needs_layout_passes=False)

NPAD = 10016
ROWS_PER_TILE = NPAD // NS
ZB = 313
ZCOPIES = 2


def _sc_message_layer(xh, src2d, dst2d, w2d):
    nblocks = src2d.shape[0]
    nb_per_tile = nblocks // NS
    mesh = plsc.VectorSubcoreMesh(core_axis_name="c", subcore_axis_name="s")

    @functools.partial(
        pl.kernel,
        out_type=jax.ShapeDtypeStruct((NC, NPAD, DH), jnp.float32),
        mesh=mesh,
        compiler_params=_SC_PARAMS,
        scratch_types=[
            pltpu.VMEM((nb_per_tile, EB), jnp.int32),
            pltpu.VMEM((nb_per_tile, EB), jnp.int32),
            pltpu.VMEM((nb_per_tile, EB), jnp.float32),
            pltpu.VMEM((EB, DH), jnp.float32),
            pltpu.VMEM((ZB, DH), jnp.float32),
            pltpu.VMEM_SHARED((NPAD, DH), jnp.float32),
        ],
    )
    def sc_kernel(x_hbm, src_hbm, dst_hbm, w_hbm, out_hbm,
                  src_v, dst_v, w_v, rows_v, zero_v, aggr_sh):
        cid = lax.axis_index("c")
        sid = lax.axis_index("s")

        zvec = jnp.zeros((LANES,), jnp.float32)

        @pl.loop(0, ZB)
        def _(r):
            for c in range(DH // LANES):
                zero_v[r, pl.ds(c * LANES, LANES)] = zvec

        @pl.loop(0, ZCOPIES)
        def _(z):
            base = (sid * ZCOPIES + z) * ZB
            pltpu.sync_copy(zero_v, aggr_sh.at[pl.ds(base, ZB)])

        tile_row0 = sid * nb_per_tile
        pltpu.sync_copy(src_hbm.at[pl.ds(tile_row0, nb_per_tile)], src_v)
        pltpu.sync_copy(dst_hbm.at[pl.ds(tile_row0, nb_per_tile)], dst_v)
        pltpu.sync_copy(w_hbm.at[pl.ds(tile_row0, nb_per_tile)], w_v)
        plsc.subcore_barrier()

        @pl.loop(0, nb_per_tile)
        def _(k):
            pltpu.sync_copy(x_hbm.at[cid].at[src_v.at[k]], rows_v)
            kk = jnp.full((LANES,), k, dtype=jnp.int32)

            @pl.loop(0, EB)
            def _(i):
                ii = jnp.full((LANES,), i, dtype=jnp.int32)
                wb = plsc.load_gather(w_v, [kk, ii])
                for c in range(DH // LANES):
                    sl = (i, pl.ds(c * LANES, LANES))
                    rows_v[sl] = jnp.maximum(rows_v[sl] + wb, 0.0)

            pltpu.sync_copy(rows_v, aggr_sh.at[dst_v.at[k]], add=True)

        plsc.subcore_barrier()
        out_base = sid * ROWS_PER_TILE
        pltpu.sync_copy(aggr_sh.at[pl.ds(out_base, ROWS_PER_TILE)],
                        out_hbm.at[cid, pl.ds(out_base, ROWS_PER_TILE)])

    return sc_kernel(xh, src2d, dst2d, w2d)


def _tc_dense_layer(x, p0, p1, W, b2d, act):
    m_blk = 2000
    c = W.shape[1]

    def body(x_ref, p0_ref, p1_ref, w_ref, b_ref, o_ref):
        p = jnp.concatenate([p0_ref[...], p1_ref[...]], axis=1)
        s = x_ref[...] + p
        acc = lax.dot_general(s, w_ref[...], (((1,), (0,)), ((), ())),
                              preferred_element_type=jnp.float32,
                              precision=lax.Precision.HIGHEST)
        acc = acc + b_ref[...]
        if act == "relu":
            acc = jnp.maximum(acc, 0.0)
        elif act == "softmax":
            acc = acc - jnp.max(acc, axis=-1, keepdims=True)
            acc = jnp.exp(acc)
            acc = acc / jnp.sum(acc, axis=-1, keepdims=True)
        o_ref[...] = acc

    return pl.pallas_call(
        body,
        grid=(N // m_blk,),
        in_specs=[
            pl.BlockSpec((m_blk, D), lambda i: (i, 0)),
            pl.BlockSpec((m_blk, DH), lambda i: (i, 0)),
            pl.BlockSpec((m_blk, DH), lambda i: (i, 0)),
            pl.BlockSpec((D, c), lambda i: (0, 0)),
            pl.BlockSpec((1, c), lambda i: (0, 0)),
        ],
        out_specs=pl.BlockSpec((m_blk, c), lambda i: (i, 0)),
        out_shape=jax.ShapeDtypeStruct((N, c), jnp.float32),
    )(x, p0, p1, W, b2d)


def kernel(features, edge_index, edge_weight, W0, b0, W1, b1, W2, b2):
    e = edge_index.shape[1]
    src2d = edge_index[0].astype(jnp.int32).reshape(e // EB, EB)
    dst2d = edge_index[1].astype(jnp.int32).reshape(e // EB, EB)
    w2d = edge_weight.reshape(e // EB, EB)

    x = features
    for W, b, act in ((W0, b0, "relu"), (W1, b1, "relu"), (W2, b2, "softmax")):
        xh = jnp.stack([x[:, :DH], x[:, DH:]])
        parts = _sc_message_layer(xh, src2d, dst2d, w2d)
        x = _tc_dense_layer(x, parts[0, :N], parts[1, :N], W,
                            b.reshape(1, -1), act)
    return x


# speedup vs baseline: 5.7207x; 1.7304x over previous
import dataclasses
import functools

import jax
import jax.numpy as jnp
from jax import lax
from jax.experimental import pallas as pl
from jax.experimental.pallas import tpu as pltpu
from jax.experimental.pallas import tpu_sc as plsc

N = 10000
D = 128
DH = D // 2
NC = 2
NS = 16
LANES = 16
EB = 80
NBUF = 2

_SC_PARAMS = pltpu.CompilerParams(use_tc_tiling_on_sc=False)
if "needs_layout_passes" in pltpu.CompilerParams.__dataclass_fields__:
    _SC_PARAMS = dataclasses.replace(_SC_PARAMS, needs_layout_passes=False)

NPAD = 10016
ROWS_PER_TILE = NPAD // NS
ZB = 313
ZCOPIES = 2


def _sc_message_layer(xh, src2d, dst2d, w2d):
    nblocks = src2d.shape[0]
    nb = nblocks // NS
    assert nb % 2 == 0 and nb >= 4
    mesh = plsc.VectorSubcoreMesh(core_axis_name="c", subcore_axis_name="s")

    @functools.partial(
        pl.kernel,
        out_type=jax.ShapeDtypeStruct((NC, NPAD, DH), jnp.float32),
        mesh=mesh,
        compiler_params=_SC_PARAMS,
        scratch_types=[
            pltpu.VMEM((nb, EB), jnp.int32),
            pltpu.VMEM((nb, EB), jnp.int32),
            pltpu.VMEM((nb, EB), jnp.float32),
            pltpu.VMEM((NBUF, EB, DH), jnp.float32),
            pltpu.VMEM((ZB, DH), jnp.float32),
            pltpu.VMEM_SHARED((NPAD, DH), jnp.float32),
        ] + [pltpu.SemaphoreType.DMA] * (2 * NBUF),
    )
    def sc_kernel(x_hbm, src_hbm, dst_hbm, w_hbm, out_hbm,
                  src_v, dst_v, w_v, rows_v, zero_v, aggr_sh, *sems):
        gsem = sems[:NBUF]
        ssem = sems[NBUF:]
        cid = lax.axis_index("c")
        sid = lax.axis_index("s")
        x_view = x_hbm.at[cid]

        zvec = jnp.zeros((LANES,), jnp.float32)

        @pl.loop(0, ZB)
        def _(r):
            for c in range(DH // LANES):
                zero_v[r, pl.ds(c * LANES, LANES)] = zvec

        @pl.loop(0, ZCOPIES)
        def _(z):
            base = (sid * ZCOPIES + z) * ZB
            pltpu.sync_copy(zero_v, aggr_sh.at[pl.ds(base, ZB)])

        tile_row0 = sid * nb
        pltpu.sync_copy(src_hbm.at[pl.ds(tile_row0, nb)], src_v)
        pltpu.sync_copy(dst_hbm.at[pl.ds(tile_row0, nb)], dst_v)
        pltpu.sync_copy(w_hbm.at[pl.ds(tile_row0, nb)], w_v)
        plsc.subcore_barrier()

        def start_gather(k, b):
            pltpu.async_copy(x_view.at[src_v.at[k]], rows_v.at[b], gsem[b])

        def wait_gather(k, b):
            pltpu.make_async_copy(x_view.at[src_v.at[k]], rows_v.at[b],
                                  gsem[b]).wait()

        def start_scatter(k, b):
            pltpu.async_copy(rows_v.at[b], aggr_sh.at[dst_v.at[k]], ssem[b],
                             add=True)

        def wait_scatter(k, b):
            pltpu.make_async_copy(rows_v.at[b], aggr_sh.at[dst_v.at[k]],
                                  ssem[b]).wait()

        def compute(k, b):
            kk = jnp.full((LANES,), k, dtype=jnp.int32)

            @pl.loop(0, EB, unroll=8)
            def _(i):
                ii = jnp.full((LANES,), i, dtype=jnp.int32)
                wb = plsc.load_gather(w_v, [kk, ii])
                for c in range(DH // LANES):
                    sl = (b, i, pl.ds(c * LANES, LANES))
                    rows_v[sl] = jnp.maximum(rows_v[sl] + wb, 0.0)

        def step(k, b, drain, prefetch):
            if drain:
                wait_scatter(k - 1, 1 - b)
            if prefetch:
                start_gather(k + 1, 1 - b)
            wait_gather(k, b)
            compute(k, b)
            start_scatter(k, b)

        start_gather(0, 0)
        step(jnp.int32(0), 0, False, True)

        @pl.loop(1, nb - 1, step=2)
        def _(k0):
            for j in range(2):
                step(k0 + j, (1 + j) % NBUF, True, True)

        step(jnp.int32(nb - 1), (nb - 1) % NBUF, True, False)
        wait_scatter(nb - 1, (nb - 1) % NBUF)

        plsc.subcore_barrier()
        out_base = sid * ROWS_PER_TILE
        pltpu.sync_copy(aggr_sh.at[pl.ds(out_base, ROWS_PER_TILE)],
                        out_hbm.at[cid, pl.ds(out_base, ROWS_PER_TILE)])

    return sc_kernel(xh, src2d, dst2d, w2d)


def _tc_dense_layer(x, p0, p1, W, b2d, act):
    m_blk = 2000
    c = W.shape[1]

    def body(x_ref, p0_ref, p1_ref, w_ref, b_ref, o_ref):
        p = jnp.concatenate([p0_ref[...], p1_ref[...]], axis=1)
        s = x_ref[...] + p
        acc = lax.dot_general(s, w_ref[...], (((1,), (0,)), ((), ())),
                              preferred_element_type=jnp.float32,
                              precision=lax.Precision.HIGHEST)
        acc = acc + b_ref[...]
        if act == "relu":
            acc = jnp.maximum(acc, 0.0)
        elif act == "softmax":
            acc = acc - jnp.max(acc, axis=-1, keepdims=True)
            acc = jnp.exp(acc)
            acc = acc / jnp.sum(acc, axis=-1, keepdims=True)
        o_ref[...] = acc

    return pl.pallas_call(
        body,
        grid=(N // m_blk,),
        in_specs=[
            pl.BlockSpec((m_blk, D), lambda i: (i, 0)),
            pl.BlockSpec((m_blk, DH), lambda i: (i, 0)),
            pl.BlockSpec((m_blk, DH), lambda i: (i, 0)),
            pl.BlockSpec((D, c), lambda i: (0, 0)),
            pl.BlockSpec((1, c), lambda i: (0, 0)),
        ],
        out_specs=pl.BlockSpec((m_blk, c), lambda i: (i, 0)),
        out_shape=jax.ShapeDtypeStruct((N, c), jnp.float32),
    )(x, p0, p1, W, b2d)


def kernel(features, edge_index, edge_weight, W0, b0, W1, b1, W2, b2):
    e = edge_index.shape[1]
    src2d = edge_index[0].astype(jnp.int32).reshape(e // EB, EB)
    dst2d = edge_index[1].astype(jnp.int32).reshape(e // EB, EB)
    w2d = edge_weight.reshape(e // EB, EB)

    x = features
    for W, b, act in ((W0, b0, "relu"), (W1, b1, "relu"), (W2, b2, "softmax")):
        xh = jnp.stack([x[:, :DH], x[:, DH:]])
        parts = _sc_message_layer(xh, src2d, dst2d, w2d)
        x = _tc_dense_layer(x, parts[0, :N], parts[1, :N], W,
                            b.reshape(1, -1), act)
    return x
